# Initial kernel scaffold; baseline (speedup 1.0000x reference)
import jax
import jax.numpy as jnp
from jax.experimental import pallas as pl


def kernel(x, edge_index, batch, Wl0, bl0, Wr0, Wl1, bl1, Wr1, Wl2, bl2, Wr2, Wh, bh):
    raise NotImplementedError("write your pallas kernel here")


# speedup vs baseline: 6.6787x; 6.6787x over previous
import functools

import jax
import jax.numpy as jnp
from jax import lax
from jax.experimental import pallas as pl
from jax.experimental.pallas import tpu as pltpu
from jax.experimental.pallas import tpu_sc as plsc

N = 10000
E = 320000
DIN = 128
H = 64
G = 32

NPAD = 10240
BLK = 1024
NBLK = NPAD // BLK
ROWS_PER_TILE = NPAD // 16
NC = 2
NS = 16
NW = NC * NS
CB = 128
CHUNKS = 79
EPAD = NW * CHUNKS * CB
DUMMY_DST = N


def _sc_agg_body(with_deg, *refs):
    if with_deg:
        (p_hbm, src3, dst3, zbig, zsm, ones_hbm,
         agg_out, deg_out,
         src_v, dst_v, rows_v, ones_v, sem,
         sh_agg, sh_deg) = refs
    else:
        (p_hbm, src3, dst3, zbig,
         agg_out,
         src_v, dst_v, rows_v, sem,
         sh_agg) = refs

    cid = lax.axis_index("c")
    sid = lax.axis_index("s")
    wid = sid * NC + cid

    base = sid * ROWS_PER_TILE
    pltpu.sync_copy(zbig.at[pl.ds(base, ROWS_PER_TILE)],
                    sh_agg.at[pl.ds(base, ROWS_PER_TILE)])
    if with_deg:
        pltpu.sync_copy(zsm.at[pl.ds(base, ROWS_PER_TILE)],
                        sh_deg.at[pl.ds(base, ROWS_PER_TILE)])
        pltpu.sync_copy(ones_hbm, ones_v)

    pltpu.sync_copy(src3.at[wid], src_v)
    pltpu.sync_copy(dst3.at[wid], dst_v)
    plsc.subcore_barrier()

    def step(j, carry):
        pltpu.async_copy(p_hbm.at[src_v.at[j]], rows_v, sem).wait()
        pltpu.sync_copy(rows_v, sh_agg.at[dst_v.at[j]], add=True)
        if with_deg:
            pltpu.sync_copy(ones_v, sh_deg.at[dst_v.at[j]], add=True)
        return carry

    lax.fori_loop(0, CHUNKS, step, 0)
    plsc.subcore_barrier()

    pltpu.sync_copy(sh_agg.at[pl.ds(base, ROWS_PER_TILE)],
                    agg_out.at[cid, pl.ds(base, ROWS_PER_TILE)])
    if with_deg:
        pltpu.sync_copy(sh_deg.at[pl.ds(base, ROWS_PER_TILE)],
                        deg_out.at[cid, pl.ds(base, ROWS_PER_TILE)])


def _make_sc_agg(with_deg):
    mesh = plsc.VectorSubcoreMesh(core_axis_name="c", subcore_axis_name="s")
    if with_deg:
        out_type = (jax.ShapeDtypeStruct((NC, NPAD, H), jnp.float32),
                    jax.ShapeDtypeStruct((NC, NPAD, 16), jnp.float32))
        scratch = [
            pltpu.VMEM((CHUNKS, CB), jnp.int32),
            pltpu.VMEM((CHUNKS, CB), jnp.int32),
            pltpu.VMEM((CB, H), jnp.float32),
            pltpu.VMEM((CB, 16), jnp.float32),
            pltpu.SemaphoreType.DMA,
            pltpu.VMEM_SHARED((NPAD, H), jnp.float32),
            pltpu.VMEM_SHARED((NPAD, 16), jnp.float32),
        ]
    else:
        out_type = jax.ShapeDtypeStruct((NC, NPAD, H), jnp.float32)
        scratch = [
            pltpu.VMEM((CHUNKS, CB), jnp.int32),
            pltpu.VMEM((CHUNKS, CB), jnp.int32),
            pltpu.VMEM((CB, H), jnp.float32),
            pltpu.SemaphoreType.DMA,
            pltpu.VMEM_SHARED((NPAD, H), jnp.float32),
        ]
    return pl.kernel(
        functools.partial(_sc_agg_body, with_deg),
        out_type=out_type,
        mesh=mesh,
        scratch_types=scratch,
        compiler_params=pltpu.CompilerParams(use_tc_tiling_on_sc=False),
    )


def _mm_body(x_ref, w_ref, p_ref, y_ref):
    r = jnp.dot(x_ref[...], w_ref[...], preferred_element_type=jnp.float32)
    p_ref[...] = r[:, :H]
    y_ref[...] = r[:, H:]


def _mid_body(agg_ref, deg_ref, y_ref, b_ref, w_ref, p_ref, y2_ref):
    aggs = agg_ref[0] + agg_ref[1]
    deg = deg_ref[0, :, 0:1] + deg_ref[1, :, 0:1]
    invd = 1.0 / jnp.maximum(deg, 1.0)
    h = jnp.maximum(aggs * invd + b_ref[...] + y_ref[...], 0.0)
    r = jnp.dot(h, w_ref[...], preferred_element_type=jnp.float32)
    p_ref[...] = r[:, :H]
    y2_ref[...] = r[:, H:]


def _pool_body(agg_ref, deg_ref, y_ref, b_ref, batch_ref, wh_ref, bh_ref,
               out_ref, acc_s, acc_c):
    i = pl.program_id(0)
    aggs = agg_ref[0] + agg_ref[1]
    deg = deg_ref[0, :, 0:1] + deg_ref[1, :, 0:1]
    invd = 1.0 / jnp.maximum(deg, 1.0)
    h = aggs * invd + b_ref[...] + y_ref[...]
    bt = batch_ref[0, 0, :]
    onehot = (bt[:, None] == lax.broadcasted_iota(jnp.int32, (1, G), 1)
              ).astype(jnp.float32)
    ps = lax.dot_general(onehot, h, (((0,), (0,)), ((), ())),
                         preferred_element_type=jnp.float32)
    cs = lax.dot_general(onehot, jnp.ones((BLK, 1), jnp.float32),
                         (((0,), (0,)), ((), ())),
                         preferred_element_type=jnp.float32)

    @pl.when(i == 0)
    def _():
        acc_s[...] = jnp.zeros_like(acc_s)
        acc_c[...] = jnp.zeros_like(acc_c)

    acc_s[...] += ps
    acc_c[...] += cs

    @pl.when(i == NBLK - 1)
    def _():
        z = acc_s[...] / jnp.maximum(acc_c[...], 1.0)
        out_ref[...] = (jnp.sum(z * wh_ref[...], axis=1, keepdims=True)
                        + bh_ref[0, 0])


def _mm_call(x_pad, w01):
    return pl.pallas_call(
        _mm_body,
        grid=(NBLK,),
        in_specs=[
            pl.BlockSpec((BLK, DIN), lambda i: (i, 0)),
            pl.BlockSpec((DIN, 2 * H), lambda i: (0, 0)),
        ],
        out_specs=[
            pl.BlockSpec((BLK, H), lambda i: (i, 0)),
            pl.BlockSpec((BLK, H), lambda i: (i, 0)),
        ],
        out_shape=[
            jax.ShapeDtypeStruct((NPAD, H), jnp.float32),
            jax.ShapeDtypeStruct((NPAD, H), jnp.float32),
        ],
    )(x_pad, w01)


def _mid_call(aggp, degp, y, b, w):
    return pl.pallas_call(
        _mid_body,
        grid=(NBLK,),
        in_specs=[
            pl.BlockSpec((NC, BLK, H), lambda i: (0, i, 0)),
            pl.BlockSpec((NC, BLK, 16), lambda i: (0, i, 0)),
            pl.BlockSpec((BLK, H), lambda i: (i, 0)),
            pl.BlockSpec((1, H), lambda i: (0, 0)),
            pl.BlockSpec((H, 2 * H), lambda i: (0, 0)),
        ],
        out_specs=[
            pl.BlockSpec((BLK, H), lambda i: (i, 0)),
            pl.BlockSpec((BLK, H), lambda i: (i, 0)),
        ],
        out_shape=[
            jax.ShapeDtypeStruct((NPAD, H), jnp.float32),
            jax.ShapeDtypeStruct((NPAD, H), jnp.float32),
        ],
    )(aggp, degp, y, b, w)


def _pool_call(aggp, degp, y, b, batch3, wh, bh):
    return pl.pallas_call(
        _pool_body,
        grid=(NBLK,),
        in_specs=[
            pl.BlockSpec((NC, BLK, H), lambda i: (0, i, 0)),
            pl.BlockSpec((NC, BLK, 16), lambda i: (0, i, 0)),
            pl.BlockSpec((BLK, H), lambda i: (i, 0)),
            pl.BlockSpec((1, H), lambda i: (0, 0)),
            pl.BlockSpec((1, 1, BLK), lambda i: (i, 0, 0)),
            pl.BlockSpec((1, H), lambda i: (0, 0)),
            pl.BlockSpec((1, 1), lambda i: (0, 0)),
        ],
        out_specs=pl.BlockSpec((G, 1), lambda i: (0, 0)),
        out_shape=jax.ShapeDtypeStruct((G, 1), jnp.float32),
        scratch_shapes=[
            pltpu.VMEM((G, H), jnp.float32),
            pltpu.VMEM((G, 1), jnp.float32),
        ],
        compiler_params=pltpu.CompilerParams(
            dimension_semantics=("arbitrary",)),
    )(aggp, degp, y, b, batch3, wh, bh)


def kernel(x, edge_index, batch, Wl0, bl0, Wr0, Wl1, bl1, Wr1, Wl2, bl2, Wr2,
           Wh, bh):
    src, dst = edge_index[0], edge_index[1]
    epad = EPAD - E
    src_p = jnp.concatenate([src, jnp.zeros((epad,), jnp.int32)])
    dst_p = jnp.concatenate([dst, jnp.full((epad,), DUMMY_DST, jnp.int32)])
    src3 = src_p.reshape(NW, CHUNKS, CB)
    dst3 = dst_p.reshape(NW, CHUNKS, CB)

    x_pad = jnp.zeros((NPAD, DIN), jnp.float32).at[:N].set(x)
    batch3 = jnp.concatenate(
        [batch, jnp.full((NPAD - N,), G, jnp.int32)]).reshape(NBLK, 1, BLK)

    zbig = jnp.zeros((NPAD, H), jnp.float32)
    zsm = jnp.zeros((NPAD, 16), jnp.float32)
    ones16 = jnp.ones((CB, 16), jnp.float32)

    w01 = jnp.concatenate([Wl0.T, Wr0.T], axis=1)
    w11 = jnp.concatenate([Wl1.T, Wr1.T], axis=1)
    w22 = jnp.concatenate([Wl2.T, Wr2.T], axis=1)
    b0 = bl0.reshape(1, H)
    b1 = bl1.reshape(1, H)
    b2 = bl2.reshape(1, H)
    wh2 = Wh.reshape(1, H)
    bh2 = bh.reshape(1, 1)

    sc_deg = _make_sc_agg(True)
    sc_plain = _make_sc_agg(False)

    p0, y0 = _mm_call(x_pad, w01)
    agg0, deg0 = sc_deg(p0, src3, dst3, zbig, zsm, ones16)
    p1, y1 = _mid_call(agg0, deg0, y0, b0, w11)
    agg1 = sc_plain(p1, src3, dst3, zbig)
    p2, y2 = _mid_call(agg1, deg0, y1, b1, w22)
    agg2 = sc_plain(p2, src3, dst3, zbig)
    out = _pool_call(agg2, deg0, y2, b2, batch3, wh2, bh2)
    return out.reshape(G)
